# baseline (speedup 1.0000x reference)
import functools

import jax
import jax.numpy as jnp
from jax import lax
from jax.experimental import pallas as pl
from jax.experimental.pallas import tpu as pltpu
from jax.experimental.pallas import tpu_sc as plsc

_NC, _NS, _L = 2, 16, 16
_NW = _NC * _NS
_ROWS = 128
_COLS = 32768
_ROWS_PER_W = _ROWS // _NW
_CHUNK = 2048
_NEG_INF = float("-inf")


def _sc_top2_body(x_hbm, out_hbm, buf, m1_ref, m2_ref, res_ref):
    wid = lax.axis_index("s") * _NC + lax.axis_index("c")
    lane = lax.iota(jnp.int32, _L)
    res_ref[...] = jnp.full((_L,), _NEG_INF, jnp.float32)
    for j in range(_ROWS_PER_W):
        r = wid * _ROWS_PER_W + j
        m1_ref[...] = jnp.full((_L,), _NEG_INF, jnp.float32)
        m2_ref[...] = jnp.full((_L,), _NEG_INF, jnp.float32)

        def chunk_body(c, _, r=r):
            pltpu.sync_copy(x_hbm.at[r, pl.ds(c * _CHUNK, _CHUNK)], buf)

            def vec_body(i, _):
                v = buf[pl.ds(i * _L, _L)]
                m1 = m1_ref[...]
                m2 = m2_ref[...]
                m2_ref[...] = jnp.maximum(m2, jnp.minimum(m1, v))
                m1_ref[...] = jnp.maximum(m1, v)
                return 0

            return lax.fori_loop(0, _CHUNK // _L, vec_body, 0)

        lax.fori_loop(0, _COLS // _CHUNK, chunk_body, 0)

        m1 = m1_ref[...]
        m2 = m2_ref[...]
        top1 = jnp.max(m1, axis=0)
        first = plsc.all_reduce_ffs(m1 == top1)
        m1_wo = jnp.where(lane == first, _NEG_INF, m1)
        top2 = jnp.maximum(jnp.max(m1_wo, axis=0), jnp.max(m2, axis=0))
        acc = res_ref[...]
        acc = jnp.where(lane == 2 * j, top1, acc)
        acc = jnp.where(lane == 2 * j + 1, top2, acc)
        res_ref[...] = acc
    pltpu.sync_copy(res_ref, out_hbm.at[pl.ds(wid * _L, _L)])


def _sc_top2(x):
    mesh = plsc.VectorSubcoreMesh(core_axis_name="c", subcore_axis_name="s",
                                  num_cores=_NC, num_subcores=_NS)
    return functools.partial(
        pl.kernel,
        out_type=jax.ShapeDtypeStruct((_NW * _L,), jnp.float32),
        mesh=mesh,
        scratch_types=[
            pltpu.VMEM((_CHUNK,), jnp.float32),
            pltpu.VMEM((_L,), jnp.float32),
            pltpu.VMEM((_L,), jnp.float32),
            pltpu.VMEM((_L,), jnp.float32),
        ],
        compiler_params=pltpu.CompilerParams(needs_layout_passes=False),
    )(_sc_top2_body)(x)


def kernel(x):
    raw = _sc_top2(x)
    values = raw.reshape(_NW, _L)[:, : 2 * _ROWS_PER_W].reshape(_ROWS, 2)
    return values[0:0, 0:1]
